# Initial kernel scaffold; baseline (speedup 1.0000x reference)
import jax
import jax.numpy as jnp
from jax.experimental import pallas as pl


def kernel(token_ids, weight):
    raise NotImplementedError("write your pallas kernel here")


# speedup vs baseline: 2.9689x; 2.9689x over previous
import functools

import jax
import jax.numpy as jnp
from jax import lax
from jax.experimental import pallas as pl
from jax.experimental.pallas import tpu as pltpu
from jax.experimental.pallas import tpu_sc as plsc

NC = 2
NS = 16
NW = NC * NS
CHUNK = 128


@functools.lru_cache(maxsize=None)
def _make_gather(n_chunks: int, d: int):
    mesh = plsc.VectorSubcoreMesh(core_axis_name="c", subcore_axis_name="s")

    @functools.partial(
        pl.kernel,
        mesh=mesh,
        out_type=jax.ShapeDtypeStruct((NW * n_chunks * CHUNK, d), jnp.float32),
        scratch_types=[
            pltpu.VMEM((n_chunks, CHUNK), jnp.int32),
            pltpu.VMEM((CHUNK, d), jnp.float32),
            pltpu.SemaphoreType.DMA,
        ],
    )
    def k(table_hbm, idx_hbm, out_hbm, idx_v, rows_v, sem):
        wid = lax.axis_index("s") * NC + lax.axis_index("c")
        base = wid * (n_chunks * CHUNK)
        pltpu.sync_copy(idx_hbm.at[wid], idx_v)

        def body(j, carry):
            pltpu.async_copy(table_hbm.at[idx_v.at[j]], rows_v, sem).wait()
            pltpu.sync_copy(rows_v, out_hbm.at[pl.ds(base + j * CHUNK, CHUNK)])
            return carry

        lax.fori_loop(0, n_chunks, body, 0)

    return k


def kernel(token_ids, weight):
    b, s = token_ids.shape
    _, d = weight.shape
    total = b * s
    n_chunks = total // (NW * CHUNK)
    idx = token_ids.reshape(NW, n_chunks, CHUNK).astype(jnp.int32)
    out = _make_gather(n_chunks, d)(weight, idx)
    return out.reshape(b, s, d)


# speedup vs baseline: 3.3425x; 1.1259x over previous
import functools

import jax
import jax.numpy as jnp
from jax import lax
from jax.experimental import pallas as pl
from jax.experimental.pallas import tpu as pltpu
from jax.experimental.pallas import tpu_sc as plsc

NC = 2
NS = 16
NW = NC * NS
CHUNK = 128


@functools.lru_cache(maxsize=None)
def _make_gather(n_chunks: int, d: int):
    mesh = plsc.VectorSubcoreMesh(core_axis_name="c", subcore_axis_name="s")

    @functools.partial(
        pl.kernel,
        mesh=mesh,
        out_type=jax.ShapeDtypeStruct((NW * n_chunks * CHUNK, d), jnp.float32),
        scratch_types=[
            pltpu.VMEM((n_chunks, CHUNK), jnp.int32),
            pltpu.VMEM((2, CHUNK, d), jnp.float32),
            pltpu.SemaphoreType.DMA,
            pltpu.SemaphoreType.DMA,
        ],
    )
    def k(table_hbm, idx_hbm, out_hbm, idx_v, bufs, sem_in, sem_out):
        wid = lax.axis_index("s") * NC + lax.axis_index("c")
        base = wid * (n_chunks * CHUNK)
        pltpu.sync_copy(idx_hbm.at[wid], idx_v)

        def gather(j, slot):
            return pltpu.make_async_copy(
                table_hbm.at[idx_v.at[j]], bufs.at[slot], sem_in)

        def put(j, slot):
            return pltpu.make_async_copy(
                bufs.at[slot], out_hbm.at[pl.ds(base + j * CHUNK, CHUNK)],
                sem_out)

        gather(0, 0).start()

        def body(j, carry):
            cur = j % 2
            nxt = (j + 1) % 2

            @pl.when(j >= 1)
            def _():
                put(j - 1, nxt).wait()

            @pl.when(j + 1 < n_chunks)
            def _():
                gather(j + 1, nxt).start()

            gather(j, cur).wait()
            put(j, cur).start()
            return carry

        lax.fori_loop(0, n_chunks, body, 0)
        put(n_chunks - 1, (n_chunks - 1) % 2).wait()

    return k


def kernel(token_ids, weight):
    b, s = token_ids.shape
    _, d = weight.shape
    total = b * s
    n_chunks = total // (NW * CHUNK)
    idx = token_ids.reshape(NW, n_chunks, CHUNK).astype(jnp.int32)
    out = _make_gather(n_chunks, d)(weight, idx)
    return out.reshape(b, s, d)


# speedup vs baseline: 3.3428x; 1.0001x over previous
import functools

import jax
import jax.numpy as jnp
from jax import lax
from jax.experimental import pallas as pl
from jax.experimental.pallas import tpu as pltpu
from jax.experimental.pallas import tpu_sc as plsc

NC = 2
NS = 16
NW = NC * NS
CHUNK = 128
NBUF = 4


@functools.lru_cache(maxsize=None)
def _make_gather(n_chunks: int, d: int):
    mesh = plsc.VectorSubcoreMesh(core_axis_name="c", subcore_axis_name="s")

    @functools.partial(
        pl.kernel,
        mesh=mesh,
        out_type=jax.ShapeDtypeStruct((NW * n_chunks * CHUNK, d), jnp.float32),
        scratch_types=[
            pltpu.VMEM((n_chunks, CHUNK), jnp.int32),
            pltpu.VMEM((NBUF, CHUNK, d), jnp.float32),
            pltpu.SemaphoreType.DMA((NBUF,)),
            pltpu.SemaphoreType.DMA((NBUF,)),
        ],
    )
    def k(table_hbm, idx_hbm, out_hbm, idx_v, bufs, sem_in, sem_out):
        wid = lax.axis_index("s") * NC + lax.axis_index("c")
        base = wid * (n_chunks * CHUNK)
        pltpu.sync_copy(idx_hbm.at[wid], idx_v)

        def gather(j, slot):
            return pltpu.make_async_copy(
                table_hbm.at[idx_v.at[j]], bufs.at[slot], sem_in.at[slot])

        def put(j, slot):
            return pltpu.make_async_copy(
                bufs.at[slot], out_hbm.at[pl.ds(base + j * CHUNK, CHUNK)],
                sem_out.at[slot])

        for b in range(NBUF - 1):
            gather(b, b).start()

        def body(j, carry):
            cur = j % NBUF
            ahead = (j + NBUF - 1) % NBUF

            @pl.when(j >= 1)
            def _():
                put(j - 1, ahead).wait()

            @pl.when(j + NBUF - 1 < n_chunks)
            def _():
                gather(j + NBUF - 1, ahead).start()

            gather(j, cur).wait()
            put(j, cur).start()
            return carry

        lax.fori_loop(0, n_chunks, body, 0)
        put(n_chunks - 1, (n_chunks - 1) % NBUF).wait()

    return k


def kernel(token_ids, weight):
    b, s = token_ids.shape
    _, d = weight.shape
    total = b * s
    n_chunks = total // (NW * CHUNK)
    idx = token_ids.reshape(NW, n_chunks, CHUNK).astype(jnp.int32)
    out = _make_gather(n_chunks, d)(weight, idx)
    return out.reshape(b, s, d)


# speedup vs baseline: 5.9563x; 1.7818x over previous
import functools

import jax
import jax.numpy as jnp
from jax import lax
from jax.experimental import pallas as pl
from jax.experimental.pallas import tpu as pltpu
from jax.experimental.pallas import tpu_sc as plsc

NC = 2
NS = 16
NW = NC * NS
TOKC = 4
NBUF = 3


@functools.lru_cache(maxsize=None)
def _make_gather(n_tok: int, s: int, d: int):
    tw = n_tok // NW
    n_chunks = tw // TOKC
    mesh = plsc.VectorSubcoreMesh(core_axis_name="c", subcore_axis_name="s")

    @functools.partial(
        pl.kernel,
        mesh=mesh,
        out_type=jax.ShapeDtypeStruct((n_tok, s, d), jnp.float32),
        scratch_types=[
            pltpu.VMEM((tw, s), jnp.int32),
            pltpu.VMEM((NBUF, TOKC, s, d), jnp.float32),
            pltpu.SemaphoreType.DMA((NBUF,)),
            pltpu.SemaphoreType.DMA((NBUF,)),
        ],
    )
    def k(table_hbm, idx_hbm, out_hbm, idx_v, bufs, sem_in, sem_out):
        wid = lax.axis_index("s") * NC + lax.axis_index("c")
        tok0 = wid * tw
        pltpu.sync_copy(idx_hbm.at[wid], idx_v)

        def gathers(c, slot):
            for t in range(TOKC):
                pltpu.make_async_copy(
                    table_hbm.at[idx_v.at[c * TOKC + t]],
                    bufs.at[slot].at[t], sem_in.at[slot]).start()

        def wait_gathers(c, slot):
            for t in range(TOKC):
                pltpu.make_async_copy(
                    table_hbm.at[idx_v.at[c * TOKC + t]],
                    bufs.at[slot].at[t], sem_in.at[slot]).wait()

        def put(c, slot):
            return pltpu.make_async_copy(
                bufs.at[slot],
                out_hbm.at[pl.ds(tok0 + c * TOKC, TOKC)],
                sem_out.at[slot])

        for b in range(NBUF - 1):
            gathers(b, b)

        def body(c, carry):
            cur = c % NBUF
            ahead = (c + NBUF - 1) % NBUF

            @pl.when(c >= 1)
            def _():
                put(c - 1, ahead).wait()

            @pl.when(c + NBUF - 1 < n_chunks)
            def _():
                gathers(c + NBUF - 1, ahead)

            wait_gathers(c, cur)
            put(c, cur).start()
            return carry

        lax.fori_loop(0, n_chunks, body, 0)
        put(n_chunks - 1, (n_chunks - 1) % NBUF).wait()

    return k


def kernel(token_ids, weight):
    b, s = token_ids.shape
    _, d = weight.shape
    idx = token_ids.reshape(NW, b // NW, s).astype(jnp.int32)
    return _make_gather(b, s, d)(weight, idx)
